# baseline (speedup 1.0000x reference)
import functools

import jax
import jax.numpy as jnp
from jax import lax
from jax.experimental import pallas as pl
from jax.experimental.pallas import tpu as pltpu
from jax.experimental.pallas import tpu_sc as plsc

VOCAB = 1000000
DIM = 64
BATCH = 16384

_info = plsc.get_sparse_core_info()
_NC, _NS, _L = _info.num_cores, _info.num_subcores, _info.num_lanes
_NW = _NC * _NS
_BPW = BATCH // _NW
_RING = 8
_UNROLL = 8
_TILE = 128
_TEW = _BPW * DIM


def _lane():
    return lax.iota(jnp.int32, _L)


def _scalar_at(ref, i):
    lane = _lane()
    chunk_base = (i >> 4) << 4
    chunk = ref[pl.ds(chunk_base, _L)]
    sel = jnp.where(lane == (i - chunk_base), chunk, 0)
    return jnp.sum(sel)


def _k1_body(vt_hbm, wt_hbm, te_hbm, vt_v, stage_v, *ring):
    wid = lax.axis_index("s") * _NC + lax.axis_index("c")
    pltpu.sync_copy(vt_hbm.at[wid], vt_v)

    bufs = ring[0:_RING]
    sems = ring[_RING:2 * _RING]
    lane = _lane()

    def fire(row, buf, sem):
        v = _scalar_at(vt_v, jnp.minimum(row, _BPW - 1))
        off = pl.multiple_of((v >> 7) << 7, _TILE)
        pltpu.async_copy(wt_hbm.at[:, pl.ds(off, _TILE)], buf, sem)
        return v & (_TILE - 1)

    def drain(buf, sem):
        pltpu.make_async_copy(wt_hbm.at[:, pl.ds(0, _TILE)], buf, sem).wait()

    cols = [fire(s, bufs[s], sems[s]) for s in range(_RING)]

    drows = [lane * 2, lane * 2 + 1, 32 + lane * 2, 33 + lane * 2]

    def body(k, carry):
        carry = list(carry)
        for s in range(_UNROLL):
            row = k * _UNROLL + s
            drain(bufs[s], sems[s])
            cvec = jnp.full((_L,), 0, jnp.int32) + carry[s]
            for kk in range(DIM // _L):
                tv = plsc.load_gather(bufs[s], [drows[kk], cvec])
                stage_v[pl.ds(row * DIM + kk * _L, _L)] = tv
            carry[s] = fire(row + _RING, bufs[s], sems[s])
        return tuple(carry)

    lax.fori_loop(0, _BPW // _UNROLL, body, tuple(cols))
    for s in range(_RING):
        drain(bufs[s], sems[s])

    pltpu.sync_copy(stage_v, te_hbm.at[wid])


def _k2_body(vc_hbm, wc_hbm, te_hbm, out_hbm, vc_v, te_v, dots_v, *ring):
    wid = lax.axis_index("s") * _NC + lax.axis_index("c")
    base = wid * _BPW
    pltpu.sync_copy(vc_hbm.at[wid], vc_v)
    pltpu.sync_copy(te_hbm.at[wid], te_v)

    bufs = ring[0:_RING]
    sems = ring[_RING:2 * _RING]
    lane = _lane()

    def fire(row, buf, sem):
        v = _scalar_at(vc_v, jnp.minimum(row, _BPW - 1))
        off = pl.multiple_of((v >> 7) << 7, _TILE)
        pltpu.async_copy(wc_hbm.at[:, pl.ds(off, _TILE)], buf, sem)
        return v & (_TILE - 1)

    def drain(buf, sem):
        pltpu.make_async_copy(wc_hbm.at[:, pl.ds(0, _TILE)], buf, sem).wait()

    cols = [fire(s, bufs[s], sems[s]) for s in range(_RING)]

    def body(k, carry):
        *colc, accv = carry
        colc = list(colc)
        himask = jnp.full((_L,), -65536, jnp.int32)
        for s in range(_UNROLL):
            row = k * _UNROLL + s
            drain(bufs[s], sems[s])
            colv = jnp.full((_L,), 0, jnp.int32) + colc[s]
            acc = jnp.zeros((_L,), jnp.float32)
            for kk in range(2):
                rows16 = lane + kk * _L
                packed = plsc.load_gather(bufs[s], [rows16, colv])
                lo = plsc.bitcast(jnp.left_shift(packed, 16), jnp.float32)
                hi = plsc.bitcast(packed & himask, jnp.float32)
                tv_ev = te_v[pl.ds(row * DIM + (2 * kk) * _L, _L)]
                tv_od = te_v[pl.ds(row * DIM + (2 * kk + 1) * _L, _L)]
                acc = acc + tv_ev * lo + tv_od * hi
            accv = jnp.where(lane == (row & (_L - 1)), jnp.sum(acc), accv)
            colc[s] = fire(row + _RING, bufs[s], sems[s])
        last = k * _UNROLL + _UNROLL - 1
        dots_v[pl.ds((last >> 4) << 4, _L)] = accv
        return tuple(colc) + (accv,)

    lax.fori_loop(0, _BPW // _UNROLL, body,
                  tuple(cols) + (jnp.zeros((_L,), jnp.float32),))
    for s in range(_RING):
        drain(bufs[s], sems[s])

    pltpu.sync_copy(dots_v, out_hbm.at[pl.ds(base, _BPW)])


@jax.jit
def kernel(target, context, W_t, W_c):
    vt = target.reshape(_NW, _BPW).astype(jnp.int32)
    vc = context.reshape(_NW, _BPW).astype(jnp.int32)
    wc_pairs = jax.lax.bitcast_convert_type(
        W_c.astype(jnp.bfloat16).reshape(VOCAB, DIM // 2, 2),
        jnp.int32).T

    mesh = plsc.VectorSubcoreMesh(core_axis_name="c", subcore_axis_name="s")
    params = pltpu.CompilerParams(
        needs_layout_passes=False, use_tc_tiling_on_sc=True)

    k1 = functools.partial(
        pl.kernel,
        out_type=jax.ShapeDtypeStruct((_NW, _TEW), jnp.float32),
        mesh=mesh,
        compiler_params=params,
        scratch_types=[
            pltpu.VMEM((_BPW,), jnp.int32),
            pltpu.VMEM((_TEW,), jnp.float32),
        ] + [pltpu.VMEM((DIM, _TILE), jnp.float32)] * _RING
          + [pltpu.SemaphoreType.DMA] * _RING,
    )(_k1_body)
    te_mid = k1(vt, W_t.T)

    k2 = functools.partial(
        pl.kernel,
        out_type=jax.ShapeDtypeStruct((BATCH,), jnp.float32),
        mesh=mesh,
        compiler_params=params,
        scratch_types=[
            pltpu.VMEM((_BPW,), jnp.int32),
            pltpu.VMEM((_TEW,), jnp.float32),
            pltpu.VMEM((_BPW,), jnp.float32),
        ] + [pltpu.VMEM((DIM // 2, _TILE), jnp.int32)] * _RING
          + [pltpu.SemaphoreType.DMA] * _RING,
    )(_k2_body)
    dots = k2(vc, wc_pairs, te_mid)
    return dots.reshape(BATCH, 1)


# speedup vs baseline: 3.7117x; 3.7117x over previous
import functools

import jax
import jax.numpy as jnp
from jax import lax
from jax.experimental import pallas as pl
from jax.experimental.pallas import tpu as pltpu
from jax.experimental.pallas import tpu_sc as plsc

VOCAB = 1000000
DIM = 64
BATCH = 16384

_info = plsc.get_sparse_core_info()
_NC, _NS, _L = _info.num_cores, _info.num_subcores, _info.num_lanes
_NW = _NC * _NS
_BPW = BATCH // _NW
_TRING = 4
_CRING = 8
_UNROLL = 8
_TILE = 128


def _sc_body(vt_hbm, vc_hbm, wt_hbm, wc_hbm, out_hbm,
             vt_v, vc_v, dots_v, *ring):
    wid = lax.axis_index("s") * _NC + lax.axis_index("c")
    base = wid * _BPW

    pltpu.sync_copy(vt_hbm.at[wid], vt_v)
    pltpu.sync_copy(vc_hbm.at[wid], vc_v)

    t_bufs = ring[0:_TRING]
    c_bufs = ring[_TRING:_TRING + _CRING]
    t_sems = ring[_TRING + _CRING:2 * _TRING + _CRING]
    c_sems = ring[2 * _TRING + _CRING:2 * _TRING + 2 * _CRING]
    lane = lax.iota(jnp.int32, _L)

    def scalar_at(ref, i):
        chunk_base = (i >> 4) << 4
        chunk = ref[pl.ds(chunk_base, _L)]
        sel = jnp.where(lane == (i - chunk_base), chunk, 0)
        return jnp.sum(sel)

    def fire(tab, vref, row, buf, sem):
        v = scalar_at(vref, jnp.minimum(row, _BPW - 1))
        off = pl.multiple_of((v >> 7) << 7, _TILE)
        pltpu.async_copy(tab.at[:, pl.ds(off, _TILE)], buf, sem)
        return v & (_TILE - 1)

    def drain(tab, buf, sem):
        pltpu.make_async_copy(tab.at[:, pl.ds(0, _TILE)], buf, sem).wait()

    tcols = [fire(wt_hbm, vt_v, s, t_bufs[s], t_sems[s])
             for s in range(_TRING)]
    ccols = [fire(wc_hbm, vc_v, s, c_bufs[s], c_sems[s])
             for s in range(_CRING)]

    def body(k, carry):
        carry = list(carry)
        tc = carry[0:_TRING]
        cc = carry[_TRING:_TRING + _CRING]
        accv = carry[-1]
        for s in range(_UNROLL):
            row = k * _UNROLL + s
            ts = s % _TRING
            drain(wt_hbm, t_bufs[ts], t_sems[ts])
            drain(wc_hbm, c_bufs[s], c_sems[s])
            ctv = jnp.full((_L,), 0, jnp.int32) + tc[ts]
            ccv = jnp.full((_L,), 0, jnp.int32) + cc[s]
            acc = jnp.zeros((_L,), jnp.float32)
            for kk in range(DIM // _L):
                rows16 = lane + kk * _L
                tv = plsc.load_gather(t_bufs[ts], [rows16, ctv])
                cv = plsc.load_gather(c_bufs[s], [rows16, ccv])
                acc = acc + tv * cv
            accv = jnp.where(lane == (row & (_L - 1)), jnp.sum(acc), accv)
            tc[ts] = fire(wt_hbm, vt_v, row + _TRING, t_bufs[ts], t_sems[ts])
            cc[s] = fire(wc_hbm, vc_v, row + _CRING, c_bufs[s], c_sems[s])
        last = k * _UNROLL + _UNROLL - 1
        dots_v[pl.ds((last >> 4) << 4, _L)] = accv
        return tuple(tc) + tuple(cc) + (accv,)

    lax.fori_loop(0, _BPW // _UNROLL, body,
                  tuple(tcols) + tuple(ccols)
                  + (jnp.zeros((_L,), jnp.float32),))

    for s in range(_TRING):
        drain(wt_hbm, t_bufs[s], t_sems[s])
    for s in range(_CRING):
        drain(wc_hbm, c_bufs[s], c_sems[s])

    pltpu.sync_copy(dots_v, out_hbm.at[pl.ds(base, _BPW)])


@jax.jit
def kernel(target, context, W_t, W_c):
    vt = target.reshape(_NW, _BPW).astype(jnp.int32)
    vc = context.reshape(_NW, _BPW).astype(jnp.int32)

    run = functools.partial(
        pl.kernel,
        out_type=jax.ShapeDtypeStruct((BATCH,), jnp.float32),
        mesh=plsc.VectorSubcoreMesh(core_axis_name="c", subcore_axis_name="s"),
        compiler_params=pltpu.CompilerParams(
            needs_layout_passes=False, use_tc_tiling_on_sc=True),
        scratch_types=[
            pltpu.VMEM((_BPW,), jnp.int32),
            pltpu.VMEM((_BPW,), jnp.int32),
            pltpu.VMEM((_BPW,), jnp.float32),
        ] + [pltpu.VMEM((DIM, _TILE), jnp.float32)] * (_TRING + _CRING)
          + [pltpu.SemaphoreType.DMA] * (_TRING + _CRING),
    )(_sc_body)
    dots = run(vt, vc, W_t.T, W_c.T)
    return dots.reshape(BATCH, 1)
